# baseline (speedup 1.0000x reference)
import functools
import jax
import jax.numpy as jnp
from jax import lax
from jax.experimental import pallas as pl
from jax.experimental.pallas import tpu as pltpu
from jax.experimental.pallas import tpu_sc as plsc

NUM_NODE = 1024
TIME_DIM = 32
ROWS = 256
LANES = 128
PAIRS_PER_STEP = 32

_NC = 2
_NS = 16
_NW = _NC * _NS


def _sc_gather(idx, table4, n_pairs):
    per_w = n_pairs // _NW
    mesh = plsc.VectorSubcoreMesh(
        core_axis_name="c", subcore_axis_name="s",
        num_cores=_NC, num_subcores=_NS)

    @functools.partial(
        pl.kernel,
        mesh=mesh,
        out_type=jax.ShapeDtypeStruct((n_pairs, LANES), jnp.float32),
        scratch_types=[
            pltpu.VMEM((per_w,), jnp.int32),
            pltpu.VMEM((per_w, LANES), jnp.float32),
            pltpu.SemaphoreType.DMA,
        ],
    )
    def k(idx_hbm, table_hbm, out_hbm, idx_v, rows_v, sem):
        wid = lax.axis_index("s") * _NC + lax.axis_index("c")
        base = wid * per_w
        pltpu.sync_copy(idx_hbm.at[pl.ds(base, per_w)], idx_v)
        pltpu.async_copy(table_hbm.at[idx_v], rows_v, sem).wait()
        pltpu.sync_copy(rows_v, out_hbm.at[pl.ds(base, per_w)])

    return k(idx, table4)


def _tc_body(rows_ref, out_ref):
    r = rows_ref[...]
    out_ref[...] = jnp.broadcast_to(r[:, None, :], (PAIRS_PER_STEP, ROWS, LANES))


def kernel(time, weekday, embedding_time):
    del weekday
    batch, seq = time.shape
    n_pairs = batch * seq
    idx = time.reshape(-1).astype(jnp.int32)
    table4 = jnp.concatenate([embedding_time] * 4, axis=1)
    rows4 = _sc_gather(idx, table4, n_pairs)

    grid = n_pairs // PAIRS_PER_STEP
    out = pl.pallas_call(
        _tc_body,
        grid=(grid,),
        in_specs=[pl.BlockSpec((PAIRS_PER_STEP, LANES), lambda i: (i, 0))],
        out_specs=pl.BlockSpec((PAIRS_PER_STEP, ROWS, LANES), lambda i: (i, 0, 0)),
        out_shape=jax.ShapeDtypeStruct((n_pairs, ROWS, LANES), jnp.float32),
    )(rows4)
    return out.reshape(batch, seq, NUM_NODE, TIME_DIM)


# speedup vs baseline: 9.2790x; 9.2790x over previous
import functools
import jax
import jax.numpy as jnp
from jax import lax
from jax.experimental import pallas as pl
from jax.experimental.pallas import tpu as pltpu
from jax.experimental.pallas import tpu_sc as plsc

NUM_NODE = 1024
TIME_DIM = 32
LANES = 128
PAIRS_PER_STEP = 32

_NC = 2
_NS = 16
_NW = _NC * _NS


def _sc_gather(idx, table4, n_pairs):
    per_w = n_pairs // _NW
    mesh = plsc.VectorSubcoreMesh(
        core_axis_name="c", subcore_axis_name="s",
        num_cores=_NC, num_subcores=_NS)

    @functools.partial(
        pl.kernel,
        mesh=mesh,
        out_type=jax.ShapeDtypeStruct((n_pairs, LANES), jnp.float32),
        scratch_types=[
            pltpu.VMEM((per_w,), jnp.int32),
            pltpu.VMEM((per_w, LANES), jnp.float32),
            pltpu.SemaphoreType.DMA,
        ],
    )
    def k(idx_hbm, table_hbm, out_hbm, idx_v, rows_v, sem):
        wid = lax.axis_index("s") * _NC + lax.axis_index("c")
        base = wid * per_w
        pltpu.sync_copy(idx_hbm.at[pl.ds(base, per_w)], idx_v)
        pltpu.async_copy(table_hbm.at[idx_v], rows_v, sem).wait()
        pltpu.sync_copy(rows_v, out_hbm.at[pl.ds(base, per_w)])

    return k(idx, table4)


def _tc_body(rows_ref, out_ref):
    r = rows_ref[...]
    rr = r[:, :TIME_DIM]
    out_ref[...] = jnp.broadcast_to(
        rr[:, :, None], (PAIRS_PER_STEP, TIME_DIM, NUM_NODE))


def kernel(time, weekday, embedding_time):
    del weekday
    batch, seq = time.shape
    n_pairs = batch * seq
    idx = time.reshape(-1).astype(jnp.int32)
    table4 = jnp.concatenate([embedding_time] * 4, axis=1)
    rows4 = _sc_gather(idx, table4, n_pairs)

    grid = n_pairs // PAIRS_PER_STEP
    out = pl.pallas_call(
        _tc_body,
        grid=(grid,),
        in_specs=[pl.BlockSpec((PAIRS_PER_STEP, LANES), lambda i: (i, 0))],
        out_specs=pl.BlockSpec(
            (PAIRS_PER_STEP, TIME_DIM, NUM_NODE), lambda i: (i, 0, 0)),
        out_shape=jax.ShapeDtypeStruct((n_pairs, TIME_DIM, NUM_NODE), jnp.float32),
    )(rows4)
    out = out.reshape(batch, seq, TIME_DIM, NUM_NODE)
    return jnp.transpose(out, (0, 1, 3, 2))
